# baseline (speedup 1.0000x reference)
import functools

import jax
import jax.numpy as jnp
from jax import lax
from jax.experimental import pallas as pl
from jax.experimental.pallas import tpu as pltpu
from jax.experimental.pallas import tpu_sc as plsc

_NBUF = 2


@functools.cache
def _make_lookup(batch, seq, V, D):
    info = plsc.get_sparse_core_info()
    nc, ns = info.num_cores, info.num_subcores
    nw = nc * ns
    lanes = info.num_lanes
    assert lanes == 16 and D == 64 and batch % (nw * 128) == 0 and seq % _NBUF == 0
    blocks_per_w = batch // (nw * 128)
    n_group = seq // _NBUF
    mesh = plsc.VectorSubcoreMesh(core_axis_name="c", subcore_axis_name="s")

    @functools.partial(
        pl.kernel,
        out_type=jax.ShapeDtypeStruct((seq, 8, batch // 128, 8, 128), jnp.float32),
        mesh=mesh,
        scratch_types=[
            pltpu.VMEM((128, seq), jnp.int32),
            pltpu.VMEM((seq, 128), jnp.int32),
            pltpu.VMEM((_NBUF, 128, D), jnp.float32),
            pltpu.VMEM((_NBUF, 8, 8, 128), jnp.float32),
            pltpu.SemaphoreType.DMA((_NBUF,)),
            pltpu.SemaphoreType.DMA((_NBUF,)),
        ],
        compiler_params=pltpu.CompilerParams(
            use_tc_tiling_on_sc=False, needs_layout_passes=False
        ),
    )
    def lookup(ids_hbm, table_hbm, out_hbm, ids_v, idsT, gbuf, tbuf, gsem, wsem):
        wid = lax.axis_index("s") * nc + lax.axis_index("c")
        iota = lax.iota(jnp.int32, 16)
        zeros = jnp.zeros((16,), jnp.int32)

        for blk in range(blocks_per_w):
            b_blk = wid * blocks_per_w + blk
            pltpu.sync_copy(ids_hbm.at[pl.ds(b_blk * 128, 128)], ids_v)

            def ids_t_body(s, c):
                col = zeros + s
                for m in range(8):
                    v = plsc.load_gather(ids_v, [iota + (m * 16), col])
                    idsT[s, pl.ds(m * 16, 16)] = v
                return c

            lax.fori_loop(0, seq, ids_t_body, 0)

            def fire_gather(s, n):
                pltpu.async_copy(table_hbm.at[idsT.at[s]], gbuf.at[n], gsem.at[n])

            for n in range(_NBUF):
                fire_gather(n, n)

            def group_body(g, c):
                s0 = g * _NBUF
                for n in range(_NBUF):
                    s = s0 + n
                    @pl.when(g > 0)
                    def _drain(n=n):
                        pltpu.make_async_copy(
                            tbuf.at[n], out_hbm.at[0, :, b_blk], wsem.at[n]
                        ).wait()

                    pltpu.make_async_copy(
                        table_hbm.at[idsT.at[s]], gbuf.at[n], gsem.at[n]
                    ).wait()

                    def e_hi_body(e_hi, c2, n=n):
                        for e_lo in range(8):
                            col = zeros + (e_hi * 8 + e_lo)
                            for m in range(8):
                                v = plsc.load_gather(
                                    gbuf.at[n], [iota + (m * 16), col]
                                )
                                tbuf[n, e_hi, e_lo, pl.ds(m * 16, 16)] = v
                        return c2

                    lax.fori_loop(0, 8, e_hi_body, 0)

                    @pl.when(s + _NBUF < seq)
                    def _next(s=s, n=n):
                        fire_gather(s + _NBUF, n)

                    pltpu.async_copy(
                        tbuf.at[n], out_hbm.at[s, :, b_blk], wsem.at[n]
                    )
                return c

            lax.fori_loop(0, n_group, group_body, 0)
            for n in range(_NBUF):
                pltpu.make_async_copy(
                    tbuf.at[n], out_hbm.at[0, :, b_blk], wsem.at[n]
                ).wait()

    return lookup


def kernel(token_ids, weights):
    batch, seq = token_ids.shape
    vocab, d = weights.shape
    ids = token_ids.astype(jnp.int32)
    out5 = _make_lookup(batch, seq, vocab, d)(ids, weights)
    return out5.transpose((2, 4, 0, 1, 3)).reshape(batch, seq, d)


# speedup vs baseline: 2.9678x; 2.9678x over previous
import functools

import jax
import jax.numpy as jnp
from jax import lax
from jax.experimental import pallas as pl
from jax.experimental.pallas import tpu as pltpu
from jax.experimental.pallas import tpu_sc as plsc

_ROWS = 2
_NBUF = 4


@functools.cache
def _make_lookup(batch, seq, V, D):
    info = plsc.get_sparse_core_info()
    nc, ns = info.num_cores, info.num_subcores
    nw = nc * ns
    rows_per_w = batch // nw
    n_super = rows_per_w // (_ROWS * _NBUF)
    n_half = n_super // 2
    assert rows_per_w == n_half * 2 * _ROWS * _NBUF
    mesh = plsc.VectorSubcoreMesh(core_axis_name="c", subcore_axis_name="s")

    @functools.partial(
        pl.kernel,
        out_type=jax.ShapeDtypeStruct((batch, seq, 128), jnp.float32),
        mesh=mesh,
        scratch_types=[
            pltpu.VMEM((2, _NBUF, _ROWS, seq), jnp.int32),
            pltpu.VMEM((_NBUF, _ROWS, seq, D), jnp.float32),
            pltpu.SemaphoreType.DMA((2, _NBUF)),
            pltpu.SemaphoreType.DMA((_NBUF,)),
            pltpu.SemaphoreType.DMA((_NBUF,)),
        ],
        compiler_params=pltpu.CompilerParams(use_tc_tiling_on_sc=False),
    )
    def lookup(ids_hbm, table_hbm, out_hbm, idx_v, rows_v, isem, gsem, wsem):
        wid = lax.axis_index("s") * nc + lax.axis_index("c")
        base = wid * rows_per_w

        def chunk_row(i, b):
            return base + (i * _NBUF + b) * _ROWS

        def prefetch_ids(i, p):
            for b in range(_NBUF):
                pltpu.async_copy(
                    ids_hbm.at[pl.ds(chunk_row(i, b), _ROWS)],
                    idx_v.at[p, b],
                    isem.at[p, b],
                )

        prefetch_ids(0, 0)

        def super_pair(j, carry):
            for p in range(2):
                i = 2 * j + p
                for b in range(_NBUF):
                    row = chunk_row(i, b)

                    @pl.when(i > 0)
                    def _drain(b=b, row=row):
                        pltpu.make_async_copy(
                            rows_v.at[b],
                            out_hbm.at[pl.ds(row, _ROWS), :, pl.ds(0, D)],
                            wsem.at[b],
                        ).wait()

                    pltpu.make_async_copy(
                        ids_hbm.at[pl.ds(row, _ROWS)], idx_v.at[p, b], isem.at[p, b]
                    ).wait()
                    for r in range(_ROWS):
                        pltpu.async_copy(
                            table_hbm.at[idx_v.at[p, b, r]], rows_v.at[b, r], gsem.at[b]
                        )
                if p == 0:
                    prefetch_ids(i + 1, 1)
                else:

                    @pl.when(j < n_half - 1)
                    def _next_ids(i=i):
                        prefetch_ids(i + 1, 0)

                for b in range(_NBUF):
                    row = chunk_row(i, b)
                    pltpu.make_async_copy(
                        table_hbm.at[idx_v.at[p, b, 0]], rows_v.at[b], gsem.at[b]
                    ).wait()
                    pltpu.async_copy(
                        rows_v.at[b],
                        out_hbm.at[pl.ds(row, _ROWS), :, pl.ds(0, D)],
                        wsem.at[b],
                    )
            return carry

        lax.fori_loop(0, n_half, super_pair, 0)
        for b in range(_NBUF):
            pltpu.make_async_copy(
                rows_v.at[b],
                out_hbm.at[pl.ds(base, _ROWS), :, pl.ds(0, D)],
                wsem.at[b],
            ).wait()

    return lookup


def kernel(token_ids, weights):
    batch, seq = token_ids.shape
    vocab, d = weights.shape
    ids = token_ids.astype(jnp.int32)
    out_pad = _make_lookup(batch, seq, vocab, d)(ids, weights)
    return out_pad[:, :, :d]
